# Initial kernel scaffold; baseline (speedup 1.0000x reference)
import jax
import jax.numpy as jnp
from jax.experimental import pallas as pl


def kernel(fillers, roles, filler_table, role_table, W_last, b_last):
    raise NotImplementedError("write your pallas kernel here")


# speedup vs baseline: 2.2627x; 2.2627x over previous
import jax
import jax.numpy as jnp
from jax import lax
from jax.experimental import pallas as pl
from jax.experimental.pallas import tpu as pltpu


def _gather_outer_kernel(S, BB, NB_IN, fil_hbm, rol_hbm, ft_hbm, rt_ref, x3_ref,
                         ft_vmem, feA, feB, reA, reB, fsm, rsm, ft_sem, idx_sem):
    c = pl.program_id(0)
    i = pl.program_id(1)
    blk = c * NB_IN + i

    @pl.when(i == 0)
    def _load_table():
        cp = pltpu.make_async_copy(ft_hbm, ft_vmem, ft_sem)
        cp.start()
        cp.wait()

    row0 = blk * BB
    cp_f = pltpu.make_async_copy(fil_hbm.at[pl.ds(row0, BB)], fsm, idx_sem)
    cp_r = pltpu.make_async_copy(rol_hbm.at[pl.ds(row0, BB)], rsm, idx_sem)
    cp_f.start()
    cp_r.start()
    cp_f.wait()
    cp_r.wait()

    def gather_b(bl, fe_t, re_t):
        for s in range(S):
            f = fsm[bl, s]
            r = rsm[bl, s]
            fe_t[s] = ft_vmem[f, 0]
            re_t[s] = rt_ref[r, 0]

    def dot_b(fe_t, re_t):
        return lax.dot_general(fe_t[...], re_t[...], (((0,), (0,)), ((), ())),
                               preferred_element_type=jnp.float32)

    def body(j, carry):
        b0 = 2 * j
        oB = dot_b(feB, reB)
        gather_b(b0, feA, reA)
        x3_ref[jnp.maximum(b0 - 1, 0)] = oB
        oA = dot_b(feA, reA)
        gather_b(b0 + 1, feB, reB)
        x3_ref[b0] = oA
        return carry

    lax.fori_loop(0, BB // 2, body, 0)
    x3_ref[BB - 1] = dot_b(feB, reB)


def _final_matmul_kernel(MB, FW, x_ref, w_ref, b_ref, o_ref):
    t = pl.program_id(1)

    @pl.when(t == 0)
    def _init():
        o_ref[...] = jnp.broadcast_to(b_ref[...], o_ref.shape)

    acc = None
    RD = x_ref.shape[2]
    for j in range(x_ref.shape[1]):
        xj = x_ref[:, j, :]
        wj = w_ref[:, j * RD:(j + 1) * RD]
        d = lax.dot_general(xj, wj, (((1,), (1,)), ((), ())),
                            preferred_element_type=jnp.float32)
        acc = d if acc is None else acc + d
    o_ref[...] = o_ref[...] + acc


def kernel(fillers, roles, filler_table, role_table, W_last, b_last):
    B, S = fillers.shape
    NF, FD = filler_table.shape
    NR, RD = role_table.shape
    FW = W_last.shape[0]

    NB_IN = 2
    BB = B // (2 * NB_IN)

    ft3 = filler_table.reshape(NF, 1, FD)
    rt3 = role_table.reshape(NR, 1, RD)
    fillers = fillers.astype(jnp.int32)
    roles = roles.astype(jnp.int32)

    x3 = pl.pallas_call(
        lambda *a: _gather_outer_kernel(S, BB, NB_IN, *a),
        out_shape=jax.ShapeDtypeStruct((B, FD, RD), jnp.float32),
        grid=(2, NB_IN),
        in_specs=[
            pl.BlockSpec(memory_space=pl.ANY),
            pl.BlockSpec(memory_space=pl.ANY),
            pl.BlockSpec(memory_space=pl.ANY),
            pl.BlockSpec((NR, 1, RD), lambda c, i: (0, 0, 0)),
        ],
        out_specs=pl.BlockSpec((BB, FD, RD), lambda c, i: (c * NB_IN + i, 0, 0)),
        scratch_shapes=[
            pltpu.VMEM((NF, 1, FD), jnp.float32),
            pltpu.VMEM((S, FD), jnp.float32),
            pltpu.VMEM((S, FD), jnp.float32),
            pltpu.VMEM((S, RD), jnp.float32),
            pltpu.VMEM((S, RD), jnp.float32),
            pltpu.SMEM((BB, S), jnp.int32),
            pltpu.SMEM((BB, S), jnp.int32),
            pltpu.SemaphoreType.DMA,
            pltpu.SemaphoreType.DMA,
        ],
        compiler_params=pltpu.CompilerParams(
            dimension_semantics=("parallel", "arbitrary"),
            vmem_limit_bytes=52 * 1024 * 1024,
        ),
        name="gather_outer",
    )(fillers, roles, ft3, rt3)

    MB = B // 2
    FBLK = 8
    NT = FD // FBLK
    out2 = pl.pallas_call(
        lambda *a: _final_matmul_kernel(MB, FW, *a),
        out_shape=jax.ShapeDtypeStruct((B, FW), jnp.float32),
        grid=(2, NT),
        in_specs=[
            pl.BlockSpec((MB, FBLK, RD), lambda m, t: (m, t, 0)),
            pl.BlockSpec((FW, FBLK * RD), lambda m, t: (0, t)),
            pl.BlockSpec((1, FW), lambda m, t: (0, 0)),
        ],
        out_specs=pl.BlockSpec((MB, FW), lambda m, t: (m, 0)),
        compiler_params=pltpu.CompilerParams(
            dimension_semantics=("parallel", "arbitrary"),
            vmem_limit_bytes=48 * 1024 * 1024,
        ),
        name="final_linear",
    )(x3, W_last, b_last.reshape(1, FW))

    return out2[None]


# speedup vs baseline: 2.4309x; 1.0743x over previous
import jax
import jax.numpy as jnp
from jax import lax
from jax.experimental import pallas as pl
from jax.experimental.pallas import tpu as pltpu


def _gather_outer_kernel(S, BB, NB_IN, LOG_NR, pk_hbm, ft_hbm, rt_ref, x3_ref,
                         ft_vmem, feA, feB, reA, reB, pA, pB,
                         ft_sem, semA, semB):
    c = pl.program_id(0)
    i = pl.program_id(1)
    blk = c * NB_IN + i

    @pl.when(i == 0)
    def _load_table():
        cp = pltpu.make_async_copy(ft_hbm, ft_vmem, ft_sem)
        cp.start()
        cp.wait()

    row0 = blk * BB
    rmask = (1 << LOG_NR) - 1

    pltpu.make_async_copy(pk_hbm.at[row0], pA, semA).start()

    def gather_b(p_sm, fe_t, re_t):
        for s in range(S):
            p = p_sm[s]
            f = lax.shift_right_logical(p, LOG_NR)
            r = p & rmask
            fe_t[s] = ft_vmem[f, 0]
            re_t[s] = rt_ref[r, 0]

    def dot_b(fe_t, re_t):
        return lax.dot_general(fe_t[...], re_t[...], (((0,), (0,)), ((), ())),
                               preferred_element_type=jnp.float32)

    def body(j, carry):
        b0 = row0 + 2 * j
        pltpu.make_async_copy(pk_hbm.at[b0 + 1], pB, semB).start()
        pltpu.make_async_copy(pk_hbm.at[b0], pA, semA).wait()
        oPrev = dot_b(feB, reB)
        gather_b(pA, feA, reA)
        x3_ref[jnp.maximum(2 * j - 1, 0)] = oPrev
        pltpu.make_async_copy(pk_hbm.at[b0 + 1], pB, semB).wait()
        oA = dot_b(feA, reA)
        gather_b(pB, feB, reB)
        x3_ref[2 * j] = oA
        nxt = jnp.minimum(b0 + 2, row0 + BB - 1)
        pltpu.make_async_copy(pk_hbm.at[nxt], pA, semA).start()
        return carry

    lax.fori_loop(0, BB // 2, body, 0)
    pltpu.make_async_copy(pk_hbm.at[row0 + BB - 1], pA, semA).wait()
    x3_ref[BB - 1] = dot_b(feB, reB)


def _final_matmul_kernel(MB, FW, x_ref, w_ref, b_ref, o_ref):
    t = pl.program_id(1)

    @pl.when(t == 0)
    def _init():
        o_ref[...] = jnp.broadcast_to(b_ref[...], o_ref.shape)

    acc = None
    RD = x_ref.shape[2]
    for j in range(x_ref.shape[1]):
        xj = x_ref[:, j, :]
        wj = w_ref[:, j * RD:(j + 1) * RD]
        d = lax.dot_general(xj, wj, (((1,), (1,)), ((), ())),
                            preferred_element_type=jnp.float32)
        acc = d if acc is None else acc + d
    o_ref[...] = o_ref[...] + acc


def kernel(fillers, roles, filler_table, role_table, W_last, b_last):
    B, S = fillers.shape
    NF, FD = filler_table.shape
    NR, RD = role_table.shape
    FW = W_last.shape[0]

    NB_IN = 2
    BB = B // (2 * NB_IN)

    ft3 = filler_table.reshape(NF, 1, FD)
    rt3 = role_table.reshape(NR, 1, RD)
    LOG_NR = NR.bit_length() - 1
    assert (1 << LOG_NR) == NR, "NUM_ROLES must be a power of two"
    packed = fillers.astype(jnp.int32) * NR + roles.astype(jnp.int32)

    x3 = pl.pallas_call(
        lambda *a: _gather_outer_kernel(S, BB, NB_IN, LOG_NR, *a),
        out_shape=jax.ShapeDtypeStruct((B, FD, RD), jnp.float32),
        grid=(2, NB_IN),
        in_specs=[
            pl.BlockSpec(memory_space=pl.ANY),
            pl.BlockSpec(memory_space=pl.ANY),
            pl.BlockSpec((NR, 1, RD), lambda c, i: (0, 0, 0)),
        ],
        out_specs=pl.BlockSpec((BB, FD, RD), lambda c, i: (c * NB_IN + i, 0, 0)),
        scratch_shapes=[
            pltpu.VMEM((NF, 1, FD), jnp.float32),
            pltpu.VMEM((S, FD), jnp.float32),
            pltpu.VMEM((S, FD), jnp.float32),
            pltpu.VMEM((S, RD), jnp.float32),
            pltpu.VMEM((S, RD), jnp.float32),
            pltpu.SMEM((S,), jnp.int32),
            pltpu.SMEM((S,), jnp.int32),
            pltpu.SemaphoreType.DMA,
            pltpu.SemaphoreType.DMA,
            pltpu.SemaphoreType.DMA,
        ],
        compiler_params=pltpu.CompilerParams(
            dimension_semantics=("parallel", "arbitrary"),
            vmem_limit_bytes=52 * 1024 * 1024,
        ),
        name="gather_outer",
    )(packed, ft3, rt3)

    MB = B // 2
    FBLK = 8
    NT = FD // FBLK
    out2 = pl.pallas_call(
        lambda *a: _final_matmul_kernel(MB, FW, *a),
        out_shape=jax.ShapeDtypeStruct((B, FW), jnp.float32),
        grid=(2, NT),
        in_specs=[
            pl.BlockSpec((MB, FBLK, RD), lambda m, t: (m, t, 0)),
            pl.BlockSpec((FW, FBLK * RD), lambda m, t: (0, t)),
            pl.BlockSpec((1, FW), lambda m, t: (0, 0)),
        ],
        out_specs=pl.BlockSpec((MB, FW), lambda m, t: (m, 0)),
        compiler_params=pltpu.CompilerParams(
            dimension_semantics=("parallel", "arbitrary"),
            vmem_limit_bytes=48 * 1024 * 1024,
        ),
        name="final_linear",
    )(x3, W_last, b_last.reshape(1, FW))

    return out2[None]


# speedup vs baseline: 3.0228x; 1.2435x over previous
import jax
import jax.numpy as jnp
from jax import lax
from jax.experimental import pallas as pl
from jax.experimental.pallas import tpu as pltpu


def _gather_outer_kernel(S, BB, LOG_NR, pk_hbm, ft_hbm, rt_ref, x3_ref,
                         ft_vmem, feA, feB, reA, reB, p0, p1, p2, p3,
                         ft_sem, sem0, sem1, sem2, sem3):
    i = pl.program_id(0)

    @pl.when(i == 0)
    def _load_table():
        cp = pltpu.make_async_copy(ft_hbm, ft_vmem, ft_sem)
        cp.start()
        cp.wait()

    row0 = i * BB
    rmask = (1 << LOG_NR) - 1
    bufs = (p0, p1, p2, p3)
    sems = (sem0, sem1, sem2, sem3)

    for k in range(4):
        pltpu.make_async_copy(pk_hbm.at[row0 + k], bufs[k], sems[k]).start()

    def gather_b(p_sm, fe_t, re_t):
        for s in range(S):
            p = p_sm[s]
            f = lax.shift_right_logical(p, LOG_NR)
            r = p & rmask
            fe_t[s] = ft_vmem[f, 0]
            re_t[s] = rt_ref[r, 0]

    def dot_b(fe_t, re_t):
        return lax.dot_general(fe_t[...], re_t[...], (((0,), (0,)), ((), ())),
                               preferred_element_type=jnp.float32)

    def body(j, carry):
        b0 = row0 + 4 * j
        for k in range(4):
            fe_c, re_c = (feA, reA) if k % 2 == 0 else (feB, reB)
            fe_p, re_p = (feB, reB) if k % 2 == 0 else (feA, reA)
            pltpu.make_async_copy(pk_hbm.at[b0 + k], bufs[k], sems[k]).wait()
            o = dot_b(fe_p, re_p)
            gather_b(bufs[k], fe_c, re_c)
            x3_ref[jnp.maximum(4 * j + k - 1, 0)] = o
            nxt = jnp.minimum(b0 + 4 + k, row0 + BB - 1)
            pltpu.make_async_copy(pk_hbm.at[nxt], bufs[k], sems[k]).start()
        return carry

    lax.fori_loop(0, BB // 4, body, 0)
    for k in range(4):
        pltpu.make_async_copy(pk_hbm.at[row0 + BB - 1], bufs[k], sems[k]).wait()
    x3_ref[BB - 1] = dot_b(feB, reB)


def _final_matmul_kernel(MB, FW, x_ref, w_ref, b_ref, o_ref):
    t = pl.program_id(1)

    @pl.when(t == 0)
    def _init():
        o_ref[...] = jnp.broadcast_to(b_ref[...], o_ref.shape)

    acc = None
    RD = x_ref.shape[2]
    for j in range(x_ref.shape[1]):
        xj = x_ref[:, j, :]
        wj = w_ref[:, j * RD:(j + 1) * RD]
        d = lax.dot_general(xj, wj, (((1,), (1,)), ((), ())),
                            preferred_element_type=jnp.float32)
        acc = d if acc is None else acc + d
    o_ref[...] = o_ref[...] + acc


def kernel(fillers, roles, filler_table, role_table, W_last, b_last):
    B, S = fillers.shape
    NF, FD = filler_table.shape
    NR, RD = role_table.shape
    FW = W_last.shape[0]

    NB = 4
    BB = B // NB

    ft3 = filler_table.reshape(NF, 1, FD)
    rt3 = role_table.reshape(NR, 1, RD)
    LOG_NR = NR.bit_length() - 1
    assert (1 << LOG_NR) == NR, "NUM_ROLES must be a power of two"
    packed = fillers.astype(jnp.int32) * NR + roles.astype(jnp.int32)

    x3 = pl.pallas_call(
        lambda *a: _gather_outer_kernel(S, BB, LOG_NR, *a),
        out_shape=jax.ShapeDtypeStruct((B, FD, RD), jnp.float32),
        grid=(NB,),
        in_specs=[
            pl.BlockSpec(memory_space=pl.ANY),
            pl.BlockSpec(memory_space=pl.ANY),
            pl.BlockSpec((NR, 1, RD), lambda i: (0, 0, 0)),
        ],
        out_specs=pl.BlockSpec((BB, FD, RD), lambda i: (i, 0, 0)),
        scratch_shapes=[
            pltpu.VMEM((NF, 1, FD), jnp.float32),
            pltpu.VMEM((S, FD), jnp.float32),
            pltpu.VMEM((S, FD), jnp.float32),
            pltpu.VMEM((S, RD), jnp.float32),
            pltpu.VMEM((S, RD), jnp.float32),
            pltpu.SMEM((S,), jnp.int32),
            pltpu.SMEM((S,), jnp.int32),
            pltpu.SMEM((S,), jnp.int32),
            pltpu.SMEM((S,), jnp.int32),
            pltpu.SemaphoreType.DMA,
            pltpu.SemaphoreType.DMA,
            pltpu.SemaphoreType.DMA,
            pltpu.SemaphoreType.DMA,
            pltpu.SemaphoreType.DMA,
        ],
        compiler_params=pltpu.CompilerParams(
            dimension_semantics=("arbitrary",),
            vmem_limit_bytes=52 * 1024 * 1024,
        ),
        name="gather_outer",
    )(packed, ft3, rt3)

    MB = B // 2
    FBLK = 8
    NT = FD // FBLK
    out2 = pl.pallas_call(
        lambda *a: _final_matmul_kernel(MB, FW, *a),
        out_shape=jax.ShapeDtypeStruct((B, FW), jnp.float32),
        grid=(2, NT),
        in_specs=[
            pl.BlockSpec((MB, FBLK, RD), lambda m, t: (m, t, 0)),
            pl.BlockSpec((FW, FBLK * RD), lambda m, t: (0, t)),
            pl.BlockSpec((1, FW), lambda m, t: (0, 0)),
        ],
        out_specs=pl.BlockSpec((MB, FW), lambda m, t: (m, 0)),
        compiler_params=pltpu.CompilerParams(
            dimension_semantics=("arbitrary", "arbitrary"),
            vmem_limit_bytes=48 * 1024 * 1024,
        ),
        name="final_linear",
    )(x3, W_last, b_last.reshape(1, FW))

    return out2[None]


# speedup vs baseline: 4.1711x; 1.3799x over previous
import jax
import jax.numpy as jnp
from jax import lax
from jax.experimental import pallas as pl
from jax.experimental.pallas import tpu as pltpu


def _gather_outer_kernel(S, BB, NR, fil_hbm, rol_ref, ft_hbm, rt_ref, x3_ref,
                         ft_vmem, feA, feB, reA, reB, pbA, pbB, p0, p1, p2, p3,
                         ft_sem, sem0, sem1, sem2, sem3):
    i = pl.program_id(0)

    @pl.when(i == 0)
    def _load_table():
        cp = pltpu.make_async_copy(ft_hbm, ft_vmem, ft_sem)
        cp.start()
        cp.wait()

    row0 = i * BB
    bufs = (p0, p1, p2, p3)
    sems = (sem0, sem1, sem2, sem3)

    for k in range(4):
        pltpu.make_async_copy(fil_hbm.at[row0 + k], bufs[k], sems[k]).start()

    iota8 = lax.broadcasted_iota(jnp.int32, (8, S), 0)

    def gather_b(p_sm, fe_t):
        for s in range(S):
            fe_t[s] = ft_vmem[p_sm[s], 0]

    def role_onehot_b(bl, pb_t, re_t):
        base = pl.multiple_of((bl >> 3) << 3, 8)
        chunk = rol_ref[pl.ds(base, 8), :]
        rvec = jnp.sum(jnp.where(iota8 == (bl & 7), chunk, 0),
                       axis=0, keepdims=True)
        d = jnp.broadcast_to(rvec, (8, S)) - iota8
        for t in range(NR // 8):
            pb_t[8 * t:8 * (t + 1), :] = jnp.where(d == 8 * t, 1.0, 0.0)
        re_t[...] = lax.dot_general(rt_ref[...], pb_t[...],
                                    (((0,), (0,)), ((), ())),
                                    preferred_element_type=jnp.float32)

    def dot_b(fe_t, re_t):
        return lax.dot_general(fe_t[...], re_t[...], (((0,), (1,)), ((), ())),
                               preferred_element_type=jnp.float32)

    def body(j, carry):
        b0 = 4 * j
        for k in range(4):
            bl = b0 + k
            fe_c, re_c, pb_c = (feA, reA, pbA) if k % 2 == 0 else (feB, reB, pbB)
            fe_p, re_p = (feB, reB) if k % 2 == 0 else (feA, reA)
            pltpu.make_async_copy(fil_hbm.at[row0 + bl], bufs[k], sems[k]).wait()
            o = dot_b(fe_p, re_p)
            gather_b(bufs[k], fe_c)
            role_onehot_b(bl, pb_c, re_c)
            x3_ref[jnp.maximum(bl - 1, 0)] = o
            nxt = jnp.minimum(row0 + bl + 4, row0 + BB - 1)
            pltpu.make_async_copy(fil_hbm.at[nxt], bufs[k], sems[k]).start()
        return carry

    lax.fori_loop(0, BB // 4, body, 0)
    for k in range(4):
        pltpu.make_async_copy(fil_hbm.at[row0 + BB - 1], bufs[k], sems[k]).wait()
    x3_ref[BB - 1] = dot_b(feB, reB)


def _final_matmul_kernel(MB, FW, x_ref, w_ref, b_ref, o_ref):
    t = pl.program_id(1)

    @pl.when(t == 0)
    def _init():
        o_ref[...] = jnp.broadcast_to(b_ref[...], o_ref.shape)

    acc = None
    RD = x_ref.shape[2]
    for j in range(x_ref.shape[1]):
        xj = x_ref[:, j, :]
        wj = w_ref[:, j * RD:(j + 1) * RD]
        d = lax.dot_general(xj, wj, (((1,), (1,)), ((), ())),
                            preferred_element_type=jnp.float32)
        acc = d if acc is None else acc + d
    o_ref[...] = o_ref[...] + acc


def kernel(fillers, roles, filler_table, role_table, W_last, b_last):
    B, S = fillers.shape
    NF, FD = filler_table.shape
    NR, RD = role_table.shape
    FW = W_last.shape[0]

    NB = 4
    BB = B // NB

    ft3 = filler_table.reshape(NF, 1, FD)
    fillers = fillers.astype(jnp.int32)
    roles = roles.astype(jnp.int32)

    x3 = pl.pallas_call(
        lambda *a: _gather_outer_kernel(S, BB, NR, *a),
        out_shape=jax.ShapeDtypeStruct((B, FD, RD), jnp.float32),
        grid=(NB,),
        in_specs=[
            pl.BlockSpec(memory_space=pl.ANY),
            pl.BlockSpec((BB, S), lambda i: (i, 0)),
            pl.BlockSpec(memory_space=pl.ANY),
            pl.BlockSpec((NR, RD), lambda i: (0, 0)),
        ],
        out_specs=pl.BlockSpec((BB, FD, RD), lambda i: (i, 0, 0)),
        scratch_shapes=[
            pltpu.VMEM((NF, 1, FD), jnp.float32),
            pltpu.VMEM((S, FD), jnp.float32),
            pltpu.VMEM((S, FD), jnp.float32),
            pltpu.VMEM((RD, S), jnp.float32),
            pltpu.VMEM((RD, S), jnp.float32),
            pltpu.VMEM((NR, S), jnp.float32),
            pltpu.VMEM((NR, S), jnp.float32),
            pltpu.SMEM((S,), jnp.int32),
            pltpu.SMEM((S,), jnp.int32),
            pltpu.SMEM((S,), jnp.int32),
            pltpu.SMEM((S,), jnp.int32),
            pltpu.SemaphoreType.DMA,
            pltpu.SemaphoreType.DMA,
            pltpu.SemaphoreType.DMA,
            pltpu.SemaphoreType.DMA,
            pltpu.SemaphoreType.DMA,
        ],
        compiler_params=pltpu.CompilerParams(
            dimension_semantics=("arbitrary",),
            vmem_limit_bytes=52 * 1024 * 1024,
        ),
        name="gather_outer",
    )(fillers, roles, ft3, role_table)

    MB = B // 2
    FBLK = 8
    NT = FD // FBLK
    out2 = pl.pallas_call(
        lambda *a: _final_matmul_kernel(MB, FW, *a),
        out_shape=jax.ShapeDtypeStruct((B, FW), jnp.float32),
        grid=(2, NT),
        in_specs=[
            pl.BlockSpec((MB, FBLK, RD), lambda m, t: (m, t, 0)),
            pl.BlockSpec((FW, FBLK * RD), lambda m, t: (0, t)),
            pl.BlockSpec((1, FW), lambda m, t: (0, 0)),
        ],
        out_specs=pl.BlockSpec((MB, FW), lambda m, t: (m, 0)),
        compiler_params=pltpu.CompilerParams(
            dimension_semantics=("arbitrary", "arbitrary"),
            vmem_limit_bytes=48 * 1024 * 1024,
        ),
        name="final_linear",
    )(x3, W_last, b_last.reshape(1, FW))

    return out2[None]


# speedup vs baseline: 4.2269x; 1.0134x over previous
import jax
import jax.numpy as jnp
from jax import lax
from jax.experimental import pallas as pl
from jax.experimental.pallas import tpu as pltpu


def _gather_outer_kernel(S, BB, NR, fil_hbm, rol_ref, ft_hbm, rt_ref, x3_ref,
                         ft_vmem, feA, feB, reA, reB, pbA, pbB, p0, p1, p2, p3,
                         ft_sem, sem0, sem1, sem2, sem3):
    i = pl.program_id(0)

    @pl.when(i == 0)
    def _load_table():
        cp = pltpu.make_async_copy(ft_hbm, ft_vmem, ft_sem)
        cp.start()
        cp.wait()

    row0 = i * BB
    bufs = (p0, p1, p2, p3)
    sems = (sem0, sem1, sem2, sem3)

    for k in range(4):
        pltpu.make_async_copy(fil_hbm.at[row0 + k], bufs[k], sems[k]).start()

    iota8 = lax.broadcasted_iota(jnp.int32, (8, S), 0)

    def gather_b(p_sm, fe_t):
        for s in range(S):
            fe_t[s] = ft_vmem[p_sm[s], 0]

    def role_onehot_b(bl, pb_t, re_t):
        base = pl.multiple_of((bl >> 3) << 3, 8)
        chunk = rol_ref[pl.ds(base, 8), :]
        rvec = pltpu.roll(chunk, -(bl & 7), axis=0)[0:1, :]
        d = jnp.broadcast_to(rvec, (8, S)) - iota8
        for t in range(NR // 8):
            pb_t[8 * t:8 * (t + 1), :] = jnp.where(d == 8 * t, 1.0, 0.0)
        re_t[...] = lax.dot_general(
            rt_ref[...], pb_t[...].astype(jnp.bfloat16),
            (((0,), (0,)), ((), ())),
            preferred_element_type=jnp.float32).astype(jnp.bfloat16)

    def dot_b(fe_t, re_t):
        return lax.dot_general(fe_t[...].astype(jnp.bfloat16), re_t[...],
                               (((0,), (1,)), ((), ())),
                               preferred_element_type=jnp.float32)

    def body(j, carry):
        b0 = 4 * j
        for k in range(4):
            bl = b0 + k
            fe_c, re_c, pb_c = (feA, reA, pbA) if k % 2 == 0 else (feB, reB, pbB)
            fe_p, re_p = (feB, reB) if k % 2 == 0 else (feA, reA)
            pltpu.make_async_copy(fil_hbm.at[row0 + bl], bufs[k], sems[k]).wait()
            o = dot_b(fe_p, re_p)
            gather_b(bufs[k], fe_c)
            role_onehot_b(bl, pb_c, re_c)
            x3_ref[jnp.maximum(bl - 1, 0)] = o
            nxt = jnp.minimum(row0 + bl + 4, row0 + BB - 1)
            pltpu.make_async_copy(fil_hbm.at[nxt], bufs[k], sems[k]).start()
        return carry

    lax.fori_loop(0, BB // 4, body, 0)
    for k in range(4):
        pltpu.make_async_copy(fil_hbm.at[row0 + BB - 1], bufs[k], sems[k]).wait()
    x3_ref[BB - 1] = dot_b(feB, reB)


def _final_matmul_kernel(MB, FW, x_ref, w_ref, b_ref, o_ref):
    t = pl.program_id(1)

    @pl.when(t == 0)
    def _init():
        o_ref[...] = jnp.broadcast_to(b_ref[...], o_ref.shape)

    acc = None
    RD = x_ref.shape[2]
    for j in range(x_ref.shape[1]):
        xj = x_ref[:, j, :]
        wj = w_ref[:, j * RD:(j + 1) * RD]
        d = lax.dot_general(xj, wj, (((1,), (1,)), ((), ())),
                            preferred_element_type=jnp.float32)
        acc = d if acc is None else acc + d
    o_ref[...] = o_ref[...] + acc


def kernel(fillers, roles, filler_table, role_table, W_last, b_last):
    B, S = fillers.shape
    NF, FD = filler_table.shape
    NR, RD = role_table.shape
    FW = W_last.shape[0]

    NB = 4
    BB = B // NB

    ft3 = filler_table.reshape(NF, 1, FD)
    fillers = fillers.astype(jnp.int32)
    roles = roles.astype(jnp.int32)

    x3 = pl.pallas_call(
        lambda *a: _gather_outer_kernel(S, BB, NR, *a),
        out_shape=jax.ShapeDtypeStruct((B, FD, RD), jnp.float32),
        grid=(NB,),
        in_specs=[
            pl.BlockSpec(memory_space=pl.ANY),
            pl.BlockSpec((BB, S), lambda i: (i, 0)),
            pl.BlockSpec(memory_space=pl.ANY),
            pl.BlockSpec((NR, RD), lambda i: (0, 0)),
        ],
        out_specs=pl.BlockSpec((BB, FD, RD), lambda i: (i, 0, 0)),
        scratch_shapes=[
            pltpu.VMEM((NF, 1, FD), jnp.float32),
            pltpu.VMEM((S, FD), jnp.float32),
            pltpu.VMEM((S, FD), jnp.float32),
            pltpu.VMEM((RD, S), jnp.bfloat16),
            pltpu.VMEM((RD, S), jnp.bfloat16),
            pltpu.VMEM((NR, S), jnp.float32),
            pltpu.VMEM((NR, S), jnp.float32),
            pltpu.SMEM((S,), jnp.int32),
            pltpu.SMEM((S,), jnp.int32),
            pltpu.SMEM((S,), jnp.int32),
            pltpu.SMEM((S,), jnp.int32),
            pltpu.SemaphoreType.DMA,
            pltpu.SemaphoreType.DMA,
            pltpu.SemaphoreType.DMA,
            pltpu.SemaphoreType.DMA,
            pltpu.SemaphoreType.DMA,
        ],
        compiler_params=pltpu.CompilerParams(
            dimension_semantics=("arbitrary",),
            vmem_limit_bytes=52 * 1024 * 1024,
        ),
        name="gather_outer",
    )(fillers, roles, ft3, role_table.astype(jnp.bfloat16))

    MB = B // 2
    FBLK = 8
    NT = FD // FBLK
    out2 = pl.pallas_call(
        lambda *a: _final_matmul_kernel(MB, FW, *a),
        out_shape=jax.ShapeDtypeStruct((B, FW), jnp.float32),
        grid=(2, NT),
        in_specs=[
            pl.BlockSpec((MB, FBLK, RD), lambda m, t: (m, t, 0)),
            pl.BlockSpec((FW, FBLK * RD), lambda m, t: (0, t)),
            pl.BlockSpec((1, FW), lambda m, t: (0, 0)),
        ],
        out_specs=pl.BlockSpec((MB, FW), lambda m, t: (m, 0)),
        compiler_params=pltpu.CompilerParams(
            dimension_semantics=("arbitrary", "arbitrary"),
            vmem_limit_bytes=48 * 1024 * 1024,
        ),
        name="final_linear",
    )(x3, W_last, b_last.reshape(1, FW))

    return out2[None]
